# Initial kernel scaffold; baseline (speedup 1.0000x reference)
import jax
import jax.numpy as jnp
from jax.experimental import pallas as pl


def kernel(X, X_w, table):
    raise NotImplementedError("write your pallas kernel here")


# speedup vs baseline: 2.4467x; 2.4467x over previous
import functools

import jax
import jax.numpy as jnp
from jax import lax
from jax.experimental import pallas as pl
from jax.experimental.pallas import tpu as pltpu
from jax.experimental.pallas import tpu_sc as plsc

B = 16384
L = 50
LP = 64
D = 64
LANES = 16

NUM_CORES = 2
NUM_SUBCORES = 16
NW = NUM_CORES * NUM_SUBCORES

DOCS_PER_W = B // NW
CHUNK_DOCS = 16
CHUNK_ROWS = CHUNK_DOCS * L
GATHER_BLK = 80
N_BLKS = CHUNK_ROWS // GATHER_BLK
N_CHUNKS = DOCS_PER_W // CHUNK_DOCS


def _sc_kernel(table_hbm, idx_hbm, w_hbm, out_hbm,
               idx_v, w_v, rows_v, out_v, sem):
    wid = lax.axis_index("s") * NUM_CORES + lax.axis_index("c")
    doc0 = wid * DOCS_PER_W

    def chunk_body(g, _):
        row0 = (doc0 + g * CHUNK_DOCS) * L
        pltpu.sync_copy(idx_hbm.at[pl.ds(row0, CHUNK_ROWS)], idx_v)
        pltpu.sync_copy(w_hbm.at[pl.ds(doc0 + g * CHUNK_DOCS, CHUNK_DOCS)],
                        w_v)

        copies = []
        for j in range(N_BLKS):
            s = pl.ds(j * GATHER_BLK, GATHER_BLK)
            copies.append(
                pltpu.make_async_copy(table_hbm.at[idx_v.at[s]],
                                      rows_v.at[s], sem))
        for c in copies:
            c.start()
        for c in copies:
            c.wait()

        def doc_body(c, _):
            r0 = c * L
            wv = [w_v[c, pl.ds(i * LANES, LANES)] for i in range(LP // LANES)]
            acc = [jnp.zeros((LANES,), jnp.float32) for _ in range(D // LANES)]
            for l in range(L):
                w = wv[l // LANES][l % LANES]
                for k in range(D // LANES):
                    acc[k] = acc[k] + rows_v[r0 + l, pl.ds(k * LANES, LANES)] * w
            for k in range(D // LANES):
                out_v[c, pl.ds(k * LANES, LANES)] = acc[k]
            return 0

        lax.fori_loop(0, CHUNK_DOCS, doc_body, 0)
        pltpu.sync_copy(out_v, out_hbm.at[pl.ds(doc0 + g * CHUNK_DOCS,
                                                CHUNK_DOCS)])
        return 0

    lax.fori_loop(0, N_CHUNKS, chunk_body, 0)


@jax.jit
def _run(table, idx_flat, w_flat):
    mesh = plsc.VectorSubcoreMesh(core_axis_name="c", subcore_axis_name="s")
    f = pl.kernel(
        _sc_kernel,
        mesh=mesh,
        out_type=jax.ShapeDtypeStruct((B, D), jnp.float32),
        scratch_types=[
            pltpu.VMEM((CHUNK_ROWS,), jnp.int32),
            pltpu.VMEM((CHUNK_DOCS, LP), jnp.float32),
            pltpu.VMEM((CHUNK_ROWS, D), jnp.float32),
            pltpu.VMEM((CHUNK_DOCS, D), jnp.float32),
            pltpu.SemaphoreType.DMA,
        ],
        compiler_params=pltpu.CompilerParams(use_tc_tiling_on_sc=False),
    )
    return f(table, idx_flat, w_flat)


def kernel(X, X_w, table):
    idx_flat = X.astype(jnp.int32).reshape(-1)
    w_pad = jnp.pad(X_w.astype(jnp.float32), ((0, 0), (0, LP - L)))
    return _run(table, idx_flat, w_pad)


# speedup vs baseline: 2.8048x; 1.1463x over previous
import jax
import jax.numpy as jnp
from jax import lax
from jax.experimental import pallas as pl
from jax.experimental.pallas import tpu as pltpu
from jax.experimental.pallas import tpu_sc as plsc

B = 16384
L = 50
LP = 64
D = 64
LANES = 16

NUM_CORES = 2
NUM_SUBCORES = 16
NW = NUM_CORES * NUM_SUBCORES

DOCS_PER_W = B // NW
CHUNK_DOCS = 16
CHUNK_ROWS = CHUNK_DOCS * L
GATHER_BLK = 80
N_BLKS = CHUNK_ROWS // GATHER_BLK
N_CHUNKS = DOCS_PER_W // CHUNK_DOCS
N_PAIRS = N_CHUNKS // 2


def _sc_kernel(table_hbm, idx_hbm, w_hbm, out_hbm,
               idx_v, w_v, rows_v, out_v, sem_in, sem_w, sem_rows, sem_out):
    wid = lax.axis_index("s") * NUM_CORES + lax.axis_index("c")
    doc0 = wid * DOCS_PER_W

    def idx_copy(g, b):
        d0 = doc0 + g * CHUNK_DOCS
        return pltpu.make_async_copy(idx_hbm.at[pl.ds(d0 * L, CHUNK_ROWS)],
                                     idx_v.at[b], sem_in.at[b])

    def w_copy(g, b):
        d0 = doc0 + g * CHUNK_DOCS
        return pltpu.make_async_copy(w_hbm.at[pl.ds(d0, CHUNK_DOCS)],
                                     w_v.at[b], sem_w.at[b])

    def gather_copies(b):
        return [
            pltpu.make_async_copy(
                table_hbm.at[idx_v.at[b, pl.ds(j * GATHER_BLK, GATHER_BLK)]],
                rows_v.at[b, pl.ds(j * GATHER_BLK, GATHER_BLK)],
                sem_rows.at[b])
            for j in range(N_BLKS)
        ]

    def out_copy(g, b):
        d0 = doc0 + g * CHUNK_DOCS
        return pltpu.make_async_copy(out_v.at[b],
                                     out_hbm.at[pl.ds(d0, CHUNK_DOCS)],
                                     sem_out.at[b])

    for b in range(2):
        idx_copy(b, b).start()
        w_copy(b, b).start()
    for b in range(2):
        idx_copy(b, b).wait()
        for c in gather_copies(b):
            c.start()

    def compute(g, b):
        def doc_body(c, _):
            r0 = c * L
            wv = [w_v[b, c, pl.ds(i * LANES, LANES)]
                  for i in range(LP // LANES)]
            acc = [[jnp.zeros((LANES,), jnp.float32) for _ in range(2)]
                   for _ in range(D // LANES)]
            for l in range(L):
                w = wv[l // LANES][l % LANES]
                p = l % 2
                for k in range(D // LANES):
                    acc[k][p] = acc[k][p] + (
                        rows_v[b, r0 + l, pl.ds(k * LANES, LANES)] * w)
            for k in range(D // LANES):
                out_v[b, c, pl.ds(k * LANES, LANES)] = acc[k][0] + acc[k][1]
            return 0

        lax.fori_loop(0, CHUNK_DOCS, doc_body, 0)

    def pair_body(i, _):
        for b in range(2):
            g = 2 * i + b
            for c in gather_copies(b):
                c.wait()
            @pl.when(i < N_PAIRS - 1)
            def _():
                idx_copy(g + 2, b).start()
            @pl.when(i > 0)
            def _():
                out_copy(g - 2, b).wait()
            w_copy(g, b).wait()
            compute(g, b)
            out_copy(g, b).start()
            @pl.when(i < N_PAIRS - 1)
            def _():
                w_copy(g + 2, b).start()
                idx_copy(g + 2, b).wait()
                for c in gather_copies(b):
                    c.start()
        return 0

    lax.fori_loop(0, N_PAIRS, pair_body, 0)
    for b in range(2):
        out_copy(N_CHUNKS - 2 + b, b).wait()


@jax.jit
def _run(table, idx_flat, w_pad):
    mesh = plsc.VectorSubcoreMesh(core_axis_name="c", subcore_axis_name="s")
    f = pl.kernel(
        _sc_kernel,
        mesh=mesh,
        out_type=jax.ShapeDtypeStruct((B, D), jnp.float32),
        scratch_types=[
            pltpu.VMEM((2, CHUNK_ROWS), jnp.int32),
            pltpu.VMEM((2, CHUNK_DOCS, LP), jnp.float32),
            pltpu.VMEM((2, CHUNK_ROWS, D), jnp.float32),
            pltpu.VMEM((2, CHUNK_DOCS, D), jnp.float32),
            pltpu.SemaphoreType.DMA((2,)),
            pltpu.SemaphoreType.DMA((2,)),
            pltpu.SemaphoreType.DMA((2,)),
            pltpu.SemaphoreType.DMA((2,)),
        ],
        compiler_params=pltpu.CompilerParams(use_tc_tiling_on_sc=False),
    )
    return f(table, idx_flat, w_pad)


def kernel(X, X_w, table):
    idx_flat = X.astype(jnp.int32).reshape(-1)
    w_pad = jnp.pad(X_w.astype(jnp.float32), ((0, 0), (0, LP - L)))
    return _run(table, idx_flat, w_pad)
